# baseline (speedup 1.0000x reference)
import functools

import jax
import jax.numpy as jnp
import numpy as np
from jax import lax
from jax.experimental import pallas as pl
from jax.experimental.pallas import tpu as pltpu
from jax.experimental.pallas import tpu_sc as plsc

B, L, H = 1024, 200, 128
TOK = B * L
NC, NS = 2, 16
NW = NC * NS
TOK_PER_W = TOK // NW
T = 64
NCHUNK = TOK_PER_W // T
EPS = 1e-12
_RSQRT_MAGIC = np.int32(0x5F3759DF)


def _xlane_sum(v):
    lanes = lax.iota(jnp.int32, 16)
    for s in (8, 4, 2, 1):
        v = v + v.at[lanes ^ s].get(mode="promise_in_bounds", unique_indices=True)
    return v


def _rsqrt(x):
    i = plsc.bitcast(x, jnp.int32)
    i = _RSQRT_MAGIC - lax.shift_right_logical(i, 1)
    y = plsc.bitcast(i, jnp.float32)
    for _ in range(2):
        y = y * (1.5 - 0.5 * x * y * y)
    return y


def _sc_body(ids_all, wt, st, mt, at_, dt, ptab, gamma, beta,
             out, idx_pf, idx_cur, wbuf, pbuf, gbv,
             seg_v, mod_v, sm_v, age_v, del_v,
             sem_pf, sem_cur, sem_g, sem_out):
    wid = lax.axis_index("s") * NC + lax.axis_index("c")
    base = wid * TOK_PER_W

    pltpu.sync_copy(st, seg_v)
    pltpu.sync_copy(mt, mod_v)
    pltpu.sync_copy(at_, age_v)
    pltpu.sync_copy(dt, del_v)
    pltpu.sync_copy(gamma, gbv.at[0])
    pltpu.sync_copy(beta, gbv.at[1])
    g = [gbv[0, pl.ds(c * 16, 16)] for c in range(8)]
    bta = [gbv[1, pl.ds(c * 16, 16)] for c in range(8)]
    cols = [lax.iota(jnp.int32, 16) + c * 16 for c in range(8)]
    for s2 in range(2):
        for m in range(10):
            for c in range(8):
                sm_v[s2 * 10 + m, pl.ds(c * 16, 16)] = (
                    seg_v[s2, pl.ds(c * 16, 16)] + mod_v[m, pl.ds(c * 16, 16)])
    small = (sm_v, age_v, del_v, del_v)

    def pf_issue(i, q):
        pltpu.async_copy(ids_all.at[wid, i], idx_pf.at[q], sem_pf)

    def pf_wait():
        pltpu.make_async_copy(ids_all.at[wid, 0], idx_pf.at[0], sem_pf).wait()

    def cur_issue(i):
        pltpu.async_copy(ids_all.at[wid, i], idx_cur, sem_cur)

    def cur_wait():
        pltpu.make_async_copy(ids_all.at[wid, 0], idx_cur, sem_cur).wait()

    def gather_issue(q, p):
        pltpu.async_copy(wt.at[idx_pf.at[q, 0, pl.ds(0, T)]], wbuf.at[p], sem_g)
        pltpu.async_copy(ptab.at[idx_pf.at[q, 6, pl.ds(0, T)]], pbuf.at[p], sem_g)

    def gather_wait():
        pltpu.make_async_copy(
            wt.at[idx_pf.at[0, 0, pl.ds(0, T)]], wbuf.at[0], sem_g).wait()
        pltpu.make_async_copy(
            ptab.at[idx_pf.at[0, 6, pl.ds(0, T)]], pbuf.at[0], sem_g).wait()

    def out_issue(i, p):
        pltpu.async_copy(wbuf.at[p], out.at[pl.ds(base + i * T, T)], sem_out)

    def out_wait():
        pltpu.make_async_copy(wbuf.at[0], out.at[pl.ds(base, T)], sem_out).wait()

    cur_issue(0)
    pf_issue(0, 0)
    pf_wait()
    gather_issue(0, 0)
    pf_issue(1, 1)

    def chunk(i, carry):
        p = lax.rem(i, 2)
        q1 = lax.rem(i + 1, 2)
        gather_wait()

        @pl.when(i > 0)
        def _free_outbuf():
            out_wait()

        cur_wait()
        rv5 = [[idx_cur[1 + j, pl.ds(gg * 16, 16)] for gg in range(4)]
               for j in range(5)]
        rv = [[rv5[0][gg] * 10 + rv5[1][gg] for gg in range(4)],
              rv5[2], rv5[3], rv5[4]]

        @pl.when(i + 1 < NCHUNK)
        def _next_cur():
            cur_issue(i + 1)

        @pl.when(i + 1 < NCHUNK)
        def _next_gathers():
            pf_wait()
            gather_issue(q1, 1 - p)

        @pl.when(i + 2 < NCHUNK)
        def _next_pf():
            pf_issue(i + 2, lax.rem(i, 2))

        for gg in range(4):
            def tok(t, c2, gg=gg):
                lane = jnp.full((16,), t, jnp.int32)
                tt = gg * 16 + t
                acc = [wbuf[p, tt, pl.ds(c * 16, 16)]
                       + pbuf[p, tt, pl.ds(c * 16, 16)] for c in range(8)]
                for j in range(4):
                    row = rv[j][gg].at[lane].get(mode="promise_in_bounds")
                    for c in range(8):
                        acc[c] = acc[c] + plsc.load_gather(small[j], [row, cols[c]])
                sq = [acc[c] * acc[c] for c in range(8)]
                s4 = [acc[2 * c] + acc[2 * c + 1] for c in range(4)]
                q4 = [sq[2 * c] + sq[2 * c + 1] for c in range(4)]
                vsum = (s4[0] + s4[1]) + (s4[2] + s4[3])
                vsq = (q4[0] + q4[1]) + (q4[2] + q4[3])
                mu = _xlane_sum(vsum) * (1.0 / H)
                var = _xlane_sum(vsq) * (1.0 / H) - mu * mu
                inv = _rsqrt(var + EPS)
                for c in range(8):
                    k = inv * g[c]
                    wbuf[p, tt, pl.ds(c * 16, 16)] = acc[c] * k + (bta[c] - mu * k)
                return c2

            lax.fori_loop(0, 16, tok, 0)

        out_issue(i, p)
        return carry

    lax.fori_loop(0, NCHUNK, chunk, 0)
    out_wait()


@jax.jit
def _run(ids_all, wt, st, mt, at_, dt, ptab, gamma, beta):
    mesh = plsc.VectorSubcoreMesh(core_axis_name="c", subcore_axis_name="s")
    f = pl.kernel(
        _sc_body,
        out_type=jax.ShapeDtypeStruct((TOK, H), jnp.float32),
        mesh=mesh,
        scratch_types=[
            pltpu.VMEM((2, 7, 128), jnp.int32),
            pltpu.VMEM((7, 128), jnp.int32),
            pltpu.VMEM((2, T, H), jnp.float32),
            pltpu.VMEM((2, T, H), jnp.float32),
            pltpu.VMEM((2, H), jnp.float32),
            pltpu.VMEM((2, H), jnp.float32),
            pltpu.VMEM((10, H), jnp.float32),
            pltpu.VMEM((20, H), jnp.float32),
            pltpu.VMEM((120, H), jnp.float32),
            pltpu.VMEM((365, H), jnp.float32),
            pltpu.SemaphoreType.DMA,
            pltpu.SemaphoreType.DMA,
            pltpu.SemaphoreType.DMA,
            pltpu.SemaphoreType.DMA,
        ],
        compiler_params=pltpu.CompilerParams(needs_layout_passes=False),
    )
    return f(ids_all, wt, st, mt, at_, dt, ptab, gamma, beta)


def kernel(word_ids, modalities_ids, age_ids, delays_ids, seg_ids, posi_ids,
           NPI_ids, word_table, seg_table, mod_table, age_table, delays_table,
           posi_table, gamma, beta):
    flat = lambda x: x.reshape(-1).astype(jnp.int32)
    ids_all = jnp.stack([
        flat(word_ids), flat(seg_ids), flat(modalities_ids), flat(age_ids),
        flat(delays_ids), flat(NPI_ids), flat(posi_ids)])
    ids_all = ids_all.reshape(7, NW, NCHUNK, T).transpose(1, 2, 0, 3)
    ids_all = jnp.pad(ids_all, ((0, 0), (0, 0), (0, 0), (0, 128 - T)))
    out = _run(ids_all, word_table, seg_table, mod_table, age_table,
               delays_table, posi_table, gamma, beta)
    return out.reshape(B, L, H)


# speedup vs baseline: 1.1405x; 1.1405x over previous
import functools

import jax
import jax.numpy as jnp
import numpy as np
from jax import lax
from jax.experimental import pallas as pl
from jax.experimental.pallas import tpu as pltpu
from jax.experimental.pallas import tpu_sc as plsc

B, L, H = 1024, 200, 128
TOK = B * L
NC, NS = 2, 16
NW = NC * NS
TOK_PER_W = TOK // NW
T = 80
NCHUNK = TOK_PER_W // T
EPS = 1e-12
_RSQRT_MAGIC = np.int32(0x5F3759DF)


def _xlane_sum(v):
    lanes = lax.iota(jnp.int32, 16)
    for s in (8, 4, 2, 1):
        v = v + v.at[lanes ^ s].get(mode="promise_in_bounds", unique_indices=True)
    return v


def _rsqrt(x):
    i = plsc.bitcast(x, jnp.int32)
    i = _RSQRT_MAGIC - lax.shift_right_logical(i, 1)
    y = plsc.bitcast(i, jnp.float32)
    for _ in range(2):
        y = y * (1.5 - 0.5 * x * y * y)
    return y


def _sc_body(ids_all, wt, st, mt, at_, dt, ptab, gamma, beta,
             out, idx_pf, idx_cur, wbuf, pbuf, gbv,
             seg_v, mod_v, sm_v, age_v, del_v,
             sem_pf, sem_cur, sem_g, sem_out):
    wid = lax.axis_index("s") * NC + lax.axis_index("c")
    base = wid * TOK_PER_W

    pltpu.sync_copy(st, seg_v)
    pltpu.sync_copy(mt, mod_v)
    pltpu.sync_copy(at_, age_v)
    pltpu.sync_copy(dt, del_v)
    pltpu.sync_copy(gamma, gbv.at[0])
    pltpu.sync_copy(beta, gbv.at[1])
    g = [gbv[0, pl.ds(c * 16, 16)] for c in range(8)]
    bta = [gbv[1, pl.ds(c * 16, 16)] for c in range(8)]
    cols = [lax.iota(jnp.int32, 16) + c * 16 for c in range(8)]
    for s2 in range(2):
        for m in range(10):
            for c in range(8):
                sm_v[s2 * 10 + m, pl.ds(c * 16, 16)] = (
                    seg_v[s2, pl.ds(c * 16, 16)] + mod_v[m, pl.ds(c * 16, 16)])
    small = (sm_v, age_v, del_v, del_v)

    def pf_issue(i, q):
        pltpu.async_copy(ids_all.at[wid, i], idx_pf.at[q], sem_pf)

    def pf_wait():
        pltpu.make_async_copy(ids_all.at[wid, 0], idx_pf.at[0], sem_pf).wait()

    def cur_issue(i):
        pltpu.async_copy(ids_all.at[wid, i], idx_cur, sem_cur)

    def cur_wait():
        pltpu.make_async_copy(ids_all.at[wid, 0], idx_cur, sem_cur).wait()

    def gather_issue(q, p):
        pltpu.async_copy(wt.at[idx_pf.at[q, 0, pl.ds(0, T)]], wbuf.at[p], sem_g)
        pltpu.async_copy(ptab.at[idx_pf.at[q, 6, pl.ds(0, T)]], pbuf.at[p], sem_g)

    def gather_wait():
        pltpu.make_async_copy(
            wt.at[idx_pf.at[0, 0, pl.ds(0, T)]], wbuf.at[0], sem_g).wait()
        pltpu.make_async_copy(
            ptab.at[idx_pf.at[0, 6, pl.ds(0, T)]], pbuf.at[0], sem_g).wait()

    def out_issue(i, p):
        pltpu.async_copy(wbuf.at[p], out.at[pl.ds(base + i * T, T)], sem_out)

    def out_wait():
        pltpu.make_async_copy(wbuf.at[0], out.at[pl.ds(base, T)], sem_out).wait()

    cur_issue(0)
    pf_issue(0, 0)
    pf_wait()
    gather_issue(0, 0)
    pf_issue(1, 1)

    def chunk(i, carry):
        p = lax.rem(i, 2)
        q1 = lax.rem(i + 1, 2)
        gather_wait()

        @pl.when(i > 0)
        def _free_outbuf():
            out_wait()

        cur_wait()
        rv5 = [[idx_cur[1 + j, pl.ds(gg * 16, 16)] for gg in range(5)]
               for j in range(5)]
        rv = [[rv5[0][gg] * 10 + rv5[1][gg] for gg in range(5)],
              rv5[2], rv5[3], rv5[4]]

        @pl.when(i + 1 < NCHUNK)
        def _next_cur():
            cur_issue(i + 1)

        @pl.when(i + 1 < NCHUNK)
        def _next_gathers():
            pf_wait()
            gather_issue(q1, 1 - p)

        @pl.when(i + 2 < NCHUNK)
        def _next_pf():
            pf_issue(i + 2, lax.rem(i, 2))

        for gg in range(5):
            def tok(t, c2, gg=gg):
                lane = jnp.full((16,), t, jnp.int32)
                tt = gg * 16 + t
                acc = [wbuf[p, tt, pl.ds(c * 16, 16)]
                       + pbuf[p, tt, pl.ds(c * 16, 16)] for c in range(8)]
                for j in range(4):
                    row = rv[j][gg].at[lane].get(mode="promise_in_bounds")
                    for c in range(8):
                        acc[c] = acc[c] + plsc.load_gather(small[j], [row, cols[c]])
                sq = [acc[c] * acc[c] for c in range(8)]
                s4 = [acc[2 * c] + acc[2 * c + 1] for c in range(4)]
                q4 = [sq[2 * c] + sq[2 * c + 1] for c in range(4)]
                vsum = (s4[0] + s4[1]) + (s4[2] + s4[3])
                vsq = (q4[0] + q4[1]) + (q4[2] + q4[3])
                mu = _xlane_sum(vsum) * (1.0 / H)
                var = _xlane_sum(vsq) * (1.0 / H) - mu * mu
                inv = _rsqrt(var + EPS)
                for c in range(8):
                    k = inv * g[c]
                    wbuf[p, tt, pl.ds(c * 16, 16)] = acc[c] * k + (bta[c] - mu * k)
                return c2

            lax.fori_loop(0, 16, tok, 0, unroll=2)

        out_issue(i, p)
        return carry

    lax.fori_loop(0, NCHUNK, chunk, 0)
    out_wait()


@jax.jit
def _run(ids_all, wt, st, mt, at_, dt, ptab, gamma, beta):
    mesh = plsc.VectorSubcoreMesh(core_axis_name="c", subcore_axis_name="s")
    f = pl.kernel(
        _sc_body,
        out_type=jax.ShapeDtypeStruct((TOK, H), jnp.float32),
        mesh=mesh,
        scratch_types=[
            pltpu.VMEM((2, 7, 128), jnp.int32),
            pltpu.VMEM((7, 128), jnp.int32),
            pltpu.VMEM((2, T, H), jnp.float32),
            pltpu.VMEM((2, T, H), jnp.float32),
            pltpu.VMEM((2, H), jnp.float32),
            pltpu.VMEM((2, H), jnp.float32),
            pltpu.VMEM((10, H), jnp.float32),
            pltpu.VMEM((20, H), jnp.float32),
            pltpu.VMEM((120, H), jnp.float32),
            pltpu.VMEM((365, H), jnp.float32),
            pltpu.SemaphoreType.DMA,
            pltpu.SemaphoreType.DMA,
            pltpu.SemaphoreType.DMA,
            pltpu.SemaphoreType.DMA,
        ],
        compiler_params=pltpu.CompilerParams(needs_layout_passes=False),
    )
    return f(ids_all, wt, st, mt, at_, dt, ptab, gamma, beta)


def kernel(word_ids, modalities_ids, age_ids, delays_ids, seg_ids, posi_ids,
           NPI_ids, word_table, seg_table, mod_table, age_table, delays_table,
           posi_table, gamma, beta):
    flat = lambda x: x.reshape(-1).astype(jnp.int32)
    ids_all = jnp.stack([
        flat(word_ids), flat(seg_ids), flat(modalities_ids), flat(age_ids),
        flat(delays_ids), flat(NPI_ids), flat(posi_ids)])
    ids_all = ids_all.reshape(7, NW, NCHUNK, T).transpose(1, 2, 0, 3)
    ids_all = jnp.pad(ids_all, ((0, 0), (0, 0), (0, 0), (0, 128 - T)))
    out = _run(ids_all, word_table, seg_table, mod_table, age_table,
               delays_table, posi_table, gamma, beta)
    return out.reshape(B, L, H)
